# baseline (speedup 1.0000x reference)
import jax
import jax.numpy as jnp
from jax import lax
from jax.experimental import pallas as pl
from jax.experimental.pallas import tpu as pltpu
from jax.experimental.pallas import tpu_sc as plsc

BATCH = 16384
EMB = 32
NC = 2
NS = 16
L = 16
NW = NC * NS
BPW = BATCH // NW
CHUNK = 128
NCHUNK = BPW // CHUNK


def _lightgcn_body(user_hbm, item_hbm, negi_hbm, utab_hbm, itab_hbm,
                   pos_hbm, neg_hbm,
                   uidx_v, iidx_v, nidx_v, urows_v, irows_v, nrows_v,
                   pos_v, negs_v, sem):
    wid = lax.axis_index("s") * NC + lax.axis_index("c")
    base = wid * BPW

    pltpu.sync_copy(user_hbm.at[wid], uidx_v)
    pltpu.sync_copy(item_hbm.at[wid], iidx_v)
    pltpu.sync_copy(negi_hbm.at[wid], nidx_v)

    copies = []
    for j in range(NCHUNK):
        dst = pl.ds(j * CHUNK, CHUNK)
        copies.append(pltpu.async_copy(utab_hbm.at[uidx_v.at[j]],
                                       urows_v.at[dst], sem))
        copies.append(pltpu.async_copy(itab_hbm.at[iidx_v.at[j]],
                                       irows_v.at[dst], sem))
        copies.append(pltpu.async_copy(itab_hbm.at[nidx_v.at[j]],
                                       nrows_v.at[dst], sem))
    for c in copies:
        c.wait()

    lanes = lax.iota(jnp.int32, L)

    def chunk_body(c, carry):
        row = c * L + lanes
        accp = jnp.zeros((L,), jnp.float32)
        accn = jnp.zeros((L,), jnp.float32)
        for d in range(EMB):
            col = jnp.full((L,), d, jnp.int32)
            uv = plsc.load_gather(urows_v, [row, col])
            iv = plsc.load_gather(irows_v, [row, col])
            nv = plsc.load_gather(nrows_v, [row, col])
            accp = accp + uv * iv
            accn = accn + uv * nv
        pos_v[pl.ds(c * L, L)] = accp
        negs_v[pl.ds(c * L, L)] = accn
        return carry

    lax.fori_loop(0, BPW // L, chunk_body, 0)

    pltpu.sync_copy(pos_v, pos_hbm.at[pl.ds(base, BPW)])
    pltpu.sync_copy(negs_v, neg_hbm.at[pl.ds(base, BPW)])


def kernel(user, item, neg_item, user_table, item_table):
    user_r = user.astype(jnp.int32).reshape(NW, NCHUNK, CHUNK)
    item_r = item.astype(jnp.int32).reshape(NW, NCHUNK, CHUNK)
    negi_r = neg_item.astype(jnp.int32).reshape(NW, NCHUNK, CHUNK)

    mesh = plsc.VectorSubcoreMesh(core_axis_name="c", subcore_axis_name="s")
    fn = pl.kernel(
        _lightgcn_body,
        out_type=(jax.ShapeDtypeStruct((BATCH,), jnp.float32),
                  jax.ShapeDtypeStruct((BATCH,), jnp.float32)),
        mesh=mesh,
        compiler_params=pltpu.CompilerParams(
            needs_layout_passes=False, use_tc_tiling_on_sc=False),
        scratch_types=[
            pltpu.VMEM((NCHUNK, CHUNK), jnp.int32),
            pltpu.VMEM((NCHUNK, CHUNK), jnp.int32),
            pltpu.VMEM((NCHUNK, CHUNK), jnp.int32),
            pltpu.VMEM((BPW, EMB), jnp.float32),
            pltpu.VMEM((BPW, EMB), jnp.float32),
            pltpu.VMEM((BPW, EMB), jnp.float32),
            pltpu.VMEM((BPW,), jnp.float32),
            pltpu.VMEM((BPW,), jnp.float32),
            pltpu.SemaphoreType.DMA,
        ],
    )
    pos, neg = fn(user_r, item_r, negi_r, user_table, item_table)
    return pos, neg
